# Initial kernel scaffold; baseline (speedup 1.0000x reference)
import functools

import jax
import jax.numpy as jnp
from jax import lax
from jax.experimental import pallas as pl
from jax.experimental.pallas import tpu as pltpu
from jax.experimental.pallas import tpu_sc as plsc

N_COMS = 4
N_NODES = 10000
N_EDGES = 320000
D_FEAT = 128
NBINS = 10240
DTOT = N_COMS * NBINS

NC, NS, L = 2, 16, 16
NW = NC * NS
E_PER_W = N_EDGES // NW
CHUNK_A = 80
NCHUNK_A = E_PER_W // CHUNK_A
GROUPS_A = CHUNK_A // L
CHUNK_B = 400
NCHUNK_B = E_PER_W // CHUNK_B
GROUPS_B = CHUNK_B // L
SLICE_B = DTOT // NS

_mesh = plsc.VectorSubcoreMesh(core_axis_name="c", subcore_axis_name="s")


def _wid():
    return lax.axis_index("s") * NC + lax.axis_index("c")


@functools.partial(
    pl.kernel,
    out_type=(
        jax.ShapeDtypeStruct((N_COMS, N_EDGES), jnp.float32),
        jax.ShapeDtypeStruct((NW, DTOT), jnp.float32),
    ),
    mesh=_mesh,
    scratch_types=(
        pltpu.VMEM((CHUNK_A,), jnp.int32),
        pltpu.VMEM((CHUNK_A,), jnp.int32),
        pltpu.VMEM((CHUNK_A, D_FEAT), jnp.float32),
        pltpu.VMEM((CHUNK_A, D_FEAT), jnp.float32),
        pltpu.VMEM((N_COMS, CHUNK_A), jnp.float32),
        pltpu.VMEM((DTOT,), jnp.float32),
        pltpu.SemaphoreType.DMA,
    ),
)
def _phase_a(phi, row, col, u_out, part_out, ri_v, ci_v, rv, cv, u_v, acc, sem):
    wid = _wid()
    base_w = wid * E_PER_W
    lanes = lax.iota(jnp.int32, L)
    zero16 = jnp.zeros((L,), jnp.float32)

    def _zero(j, _):
        acc[pl.ds(j * L, L)] = zero16
        return 0

    lax.fori_loop(0, DTOT // L, _zero, 0)

    def _chunk(c, _):
        base = base_w + c * CHUNK_A
        pltpu.sync_copy(row.at[pl.ds(base, CHUNK_A)], ri_v)
        pltpu.sync_copy(col.at[pl.ds(base, CHUNK_A)], ci_v)
        cp1 = pltpu.async_copy(phi.at[ri_v], rv, sem)
        cp2 = pltpu.async_copy(phi.at[ci_v], cv, sem)
        cp1.wait()
        cp2.wait()

        def _group(g, _):
            eidx = g * L + lanes
            accs = []
            for k in range(N_COMS):
                a = zero16
                for d in range(k * 32, (k + 1) * 32):
                    dsplat = jnp.full((L,), d, jnp.int32)
                    a = a + (plsc.load_gather(rv, [eidx, dsplat])
                             * plsc.load_gather(cv, [eidx, dsplat]))
                accs.append(a)
            m = jnp.maximum(jnp.maximum(accs[0], accs[1]),
                            jnp.maximum(accs[2], accs[3]))
            ps = [jnp.exp(a - m) for a in accs]
            t = (ps[0] + ps[1]) + (ps[2] + ps[3])
            rvec = plsc.load_gather(ri_v, [eidx])
            for k in range(N_COMS):
                u = jnp.exp((ps[k] + ps[k]) / t)
                u_v[k, pl.ds(g * L, L)] = u
                plsc.addupdate_scatter(acc, [rvec + (k * NBINS)], u)
            return 0

        lax.fori_loop(0, GROUPS_A, _group, 0)
        for k in range(N_COMS):
            pltpu.sync_copy(u_v.at[k], u_out.at[k, pl.ds(base, CHUNK_A)])
        return 0

    lax.fori_loop(0, NCHUNK_A, _chunk, 0)
    pltpu.sync_copy(acc, part_out.at[wid])


@functools.partial(
    pl.kernel,
    out_type=jax.ShapeDtypeStruct((N_COMS, N_EDGES), jnp.float32),
    mesh=_mesh,
    scratch_types=(
        pltpu.VMEM((SLICE_B,), jnp.float32),
        pltpu.VMEM((SLICE_B,), jnp.float32),
        pltpu.VMEM((DTOT,), jnp.float32),
        pltpu.VMEM((CHUNK_B,), jnp.int32),
        pltpu.VMEM((N_COMS, CHUNK_B), jnp.float32),
        pltpu.VMEM((N_COMS, CHUNK_B), jnp.float32),
        pltpu.VMEM_SHARED((DTOT,), jnp.float32),
        pltpu.SemaphoreType.DMA,
    ),
)
def _phase_b(u_in, part, row, out, tmp_v, red_v, d_v, ri_v, u_v, o_v, shared, sem):
    sid = lax.axis_index("s")
    wid = _wid()
    base_w = wid * E_PER_W
    lanes = lax.iota(jnp.int32, L)
    zero16 = jnp.zeros((L,), jnp.float32)

    def _zero(j, _):
        red_v[pl.ds(j * L, L)] = zero16
        return 0

    lax.fori_loop(0, SLICE_B // L, _zero, 0)

    def _accum(p, _):
        pltpu.sync_copy(part.at[p, pl.ds(sid * SLICE_B, SLICE_B)], tmp_v)

        def _add(j, _):
            s = pl.ds(j * L, L)
            red_v[s] = red_v[s] + tmp_v[s]
            return 0

        lax.fori_loop(0, SLICE_B // L, _add, 0)
        return 0

    lax.fori_loop(0, NW, _accum, 0)
    pltpu.sync_copy(red_v, shared.at[pl.ds(sid * SLICE_B, SLICE_B)])
    plsc.subcore_barrier()
    pltpu.sync_copy(shared, d_v)

    def _chunk(c, _):
        base = base_w + c * CHUNK_B
        pltpu.sync_copy(row.at[pl.ds(base, CHUNK_B)], ri_v)
        for k in range(N_COMS):
            pltpu.sync_copy(u_in.at[k, pl.ds(base, CHUNK_B)], u_v.at[k])

        def _group(g, _):
            eidx = g * L + lanes
            rvec = plsc.load_gather(ri_v, [eidx])
            for k in range(N_COMS):
                dk = plsc.load_gather(d_v, [rvec + (k * NBINS)])
                u = u_v[k, pl.ds(g * L, L)]
                o_v[k, pl.ds(g * L, L)] = u / dk
            return 0

        lax.fori_loop(0, GROUPS_B, _group, 0)
        for k in range(N_COMS):
            pltpu.sync_copy(o_v.at[k], out.at[k, pl.ds(base, CHUNK_B)])
        return 0

    lax.fori_loop(0, NCHUNK_B, _chunk, 0)


def kernel(phi, edge_index):
    row = edge_index[0]
    col = edge_index[1]
    u, part = _phase_a(phi, row, col)
    return _phase_b(u, part, row)


# speedup vs baseline: 12.7136x; 12.7136x over previous
import functools

import jax
import jax.numpy as jnp
from jax import lax
from jax.experimental import pallas as pl
from jax.experimental.pallas import tpu as pltpu
from jax.experimental.pallas import tpu_sc as plsc

N_COMS = 4
N_NODES = 10000
N_EDGES = 320000
D_FEAT = 128
NBINS = 10240
DTOT = N_COMS * NBINS

NC, NS, L = 2, 16, 16
NW = NC * NS
E_PER_W = N_EDGES // NW
CHUNK_A = 80
NCHUNK_A = E_PER_W // CHUNK_A
GROUPS_A = CHUNK_A // L
CHUNK_B = 400
NCHUNK_B = E_PER_W // CHUNK_B
GROUPS_B = CHUNK_B // L
SLICE_B = DTOT // NS

_mesh = plsc.VectorSubcoreMesh(core_axis_name="c", subcore_axis_name="s")


def _wid():
    return lax.axis_index("s") * NC + lax.axis_index("c")


@functools.partial(
    pl.kernel,
    out_type=(
        jax.ShapeDtypeStruct((N_COMS * N_EDGES,), jnp.float32),
        jax.ShapeDtypeStruct((NW * DTOT,), jnp.float32),
    ),
    mesh=_mesh,
    compiler_params=pltpu.CompilerParams(needs_layout_passes=False),
    scratch_types=(
        pltpu.VMEM((CHUNK_A,), jnp.int32),
        pltpu.VMEM((CHUNK_A,), jnp.int32),
        pltpu.VMEM((CHUNK_A, D_FEAT), jnp.float32),
        pltpu.VMEM((CHUNK_A, D_FEAT), jnp.float32),
        pltpu.VMEM((N_COMS * CHUNK_A,), jnp.float32),
        pltpu.VMEM((DTOT,), jnp.float32),
        pltpu.SemaphoreType.DMA,
    ),
)
def _phase_a(phi, row, col, u_out, part_out, ri_v, ci_v, rv, cv, u_v, acc, sem):
    wid = _wid()
    base_w = wid * E_PER_W
    lanes = lax.iota(jnp.int32, L)
    zero16 = jnp.zeros((L,), jnp.float32)

    def _zero(j, _):
        acc[pl.ds(j * L, L)] = zero16
        return 0

    lax.fori_loop(0, DTOT // L, _zero, 0)

    def _chunk(c, _):
        base = base_w + c * CHUNK_A
        pltpu.sync_copy(row.at[pl.ds(base, CHUNK_A)], ri_v)
        pltpu.sync_copy(col.at[pl.ds(base, CHUNK_A)], ci_v)
        cp1 = pltpu.async_copy(phi.at[ri_v], rv, sem)
        cp2 = pltpu.async_copy(phi.at[ci_v], cv, sem)
        cp1.wait()
        cp2.wait()

        def _group(g, _):
            eidx = g * L + lanes
            accs = []
            for k in range(N_COMS):
                a = zero16
                for d in range(k * 32, (k + 1) * 32):
                    dsplat = jnp.full((L,), d, jnp.int32)
                    a = a + (plsc.load_gather(rv, [eidx, dsplat])
                             * plsc.load_gather(cv, [eidx, dsplat]))
                accs.append(a)
            m = jnp.maximum(jnp.maximum(accs[0], accs[1]),
                            jnp.maximum(accs[2], accs[3]))
            ps = [jnp.exp(a - m) for a in accs]
            t = (ps[0] + ps[1]) + (ps[2] + ps[3])
            rvec = plsc.load_gather(ri_v, [eidx])
            for k in range(N_COMS):
                u = jnp.exp((ps[k] + ps[k]) / t)
                u_v[pl.ds(k * CHUNK_A + g * L, L)] = u
                plsc.addupdate_scatter(acc, [rvec + (k * NBINS)], u)
            return 0

        lax.fori_loop(0, GROUPS_A, _group, 0)
        for k in range(N_COMS):
            pltpu.sync_copy(u_v.at[pl.ds(k * CHUNK_A, CHUNK_A)], u_out.at[pl.ds(k * N_EDGES + base, CHUNK_A)])
        return 0

    lax.fori_loop(0, NCHUNK_A, _chunk, 0)
    pltpu.sync_copy(acc, part_out.at[pl.ds(wid * DTOT, DTOT)])


@functools.partial(
    pl.kernel,
    out_type=jax.ShapeDtypeStruct((N_COMS * N_EDGES,), jnp.float32),
    mesh=_mesh,
    compiler_params=pltpu.CompilerParams(needs_layout_passes=False),
    scratch_types=(
        pltpu.VMEM((SLICE_B,), jnp.float32),
        pltpu.VMEM((SLICE_B,), jnp.float32),
        pltpu.VMEM((DTOT,), jnp.float32),
        pltpu.VMEM((CHUNK_B,), jnp.int32),
        pltpu.VMEM((N_COMS * CHUNK_B,), jnp.float32),
        pltpu.VMEM((N_COMS * CHUNK_B,), jnp.float32),
        pltpu.VMEM_SHARED((DTOT,), jnp.float32),
        pltpu.SemaphoreType.DMA,
    ),
)
def _phase_b(u_in, part, row, out, tmp_v, red_v, d_v, ri_v, u_v, o_v, shared, sem):
    sid = lax.axis_index("s")
    wid = _wid()
    base_w = wid * E_PER_W
    lanes = lax.iota(jnp.int32, L)
    zero16 = jnp.zeros((L,), jnp.float32)

    def _zero(j, _):
        red_v[pl.ds(j * L, L)] = zero16
        return 0

    lax.fori_loop(0, SLICE_B // L, _zero, 0)

    def _accum(p, _):
        pltpu.sync_copy(part.at[pl.ds(p * DTOT + sid * SLICE_B, SLICE_B)], tmp_v)

        def _add(j, _):
            s = pl.ds(j * L, L)
            red_v[s] = red_v[s] + tmp_v[s]
            return 0

        lax.fori_loop(0, SLICE_B // L, _add, 0)
        return 0

    lax.fori_loop(0, NW, _accum, 0)
    pltpu.sync_copy(red_v, shared.at[pl.ds(sid * SLICE_B, SLICE_B)])
    plsc.subcore_barrier()
    pltpu.sync_copy(shared, d_v)

    def _chunk(c, _):
        base = base_w + c * CHUNK_B
        pltpu.sync_copy(row.at[pl.ds(base, CHUNK_B)], ri_v)
        for k in range(N_COMS):
            pltpu.sync_copy(u_in.at[pl.ds(k * N_EDGES + base, CHUNK_B)], u_v.at[pl.ds(k * CHUNK_B, CHUNK_B)])

        def _group(g, _):
            eidx = g * L + lanes
            rvec = plsc.load_gather(ri_v, [eidx])
            for k in range(N_COMS):
                dk = plsc.load_gather(d_v, [rvec + (k * NBINS)])
                u = u_v[pl.ds(k * CHUNK_B + g * L, L)]
                o_v[pl.ds(k * CHUNK_B + g * L, L)] = u / dk
            return 0

        lax.fori_loop(0, GROUPS_B, _group, 0)
        for k in range(N_COMS):
            pltpu.sync_copy(o_v.at[pl.ds(k * CHUNK_B, CHUNK_B)], out.at[pl.ds(k * N_EDGES + base, CHUNK_B)])
        return 0

    lax.fori_loop(0, NCHUNK_B, _chunk, 0)


def kernel(phi, edge_index):
    row = edge_index[0]
    col = edge_index[1]
    u, part = _phase_a(phi, row, col)
    return _phase_b(u, part, row).reshape(N_COMS, N_EDGES)


# speedup vs baseline: 17.7115x; 1.3931x over previous
import functools

import jax
import jax.numpy as jnp
from jax import lax
from jax.experimental import pallas as pl
from jax.experimental.pallas import tpu as pltpu
from jax.experimental.pallas import tpu_sc as plsc

N_COMS = 4
N_NODES = 10000
N_EDGES = 320000
D_FEAT = 128
NBINS = 10240
DTOT = N_COMS * NBINS

NC, NS, L = 2, 16, 16
NW = NC * NS
E_PER_W = N_EDGES // NW
CHUNK_A = 80
NCHUNK_A = E_PER_W // CHUNK_A
GROUPS_A = CHUNK_A // L
UCHUNK = N_COMS * CHUNK_A
CHUNK_B = 400
NCHUNK_B = E_PER_W // CHUNK_B
SUB_B = CHUNK_B // CHUNK_A
SLICE_B = DTOT // NS

_mesh = plsc.VectorSubcoreMesh(core_axis_name="c", subcore_axis_name="s")


def _wid():
    return lax.axis_index("s") * NC + lax.axis_index("c")


@functools.partial(
    pl.kernel,
    out_type=(
        jax.ShapeDtypeStruct((N_COMS * N_EDGES,), jnp.float32),
        jax.ShapeDtypeStruct((NW * DTOT,), jnp.float32),
    ),
    mesh=_mesh,
    compiler_params=pltpu.CompilerParams(needs_layout_passes=False),
    scratch_types=(
        pltpu.VMEM((E_PER_W,), jnp.int32),
        pltpu.VMEM((E_PER_W,), jnp.int32),
        pltpu.VMEM((CHUNK_A, D_FEAT), jnp.float32),
        pltpu.VMEM((CHUNK_A, D_FEAT), jnp.float32),
        pltpu.VMEM((CHUNK_A, D_FEAT), jnp.float32),
        pltpu.VMEM((CHUNK_A, D_FEAT), jnp.float32),
        pltpu.VMEM((UCHUNK,), jnp.float32),
        pltpu.VMEM((UCHUNK,), jnp.float32),
        pltpu.VMEM((DTOT,), jnp.float32),
        pltpu.SemaphoreType.DMA,
        pltpu.SemaphoreType.DMA,
        pltpu.SemaphoreType.DMA,
        pltpu.SemaphoreType.DMA,
    ),
)
def _phase_a(phi, row, col, u_out, part_out,
             ir_v, ic_v, rv0, cv0, rv1, cv1, uv0, uv1, acc,
             sg0, sg1, su0, su1):
    wid = _wid()
    base_w = wid * E_PER_W
    ubase_w = wid * (NCHUNK_A * UCHUNK)
    lanes = lax.iota(jnp.int32, L)
    zero16 = jnp.zeros((L,), jnp.float32)

    pltpu.sync_copy(row.at[pl.ds(base_w, E_PER_W)], ir_v)
    pltpu.sync_copy(col.at[pl.ds(base_w, E_PER_W)], ic_v)

    def _zero(j, _):
        for q in range(8):
            acc[pl.ds((j * 8 + q) * L, L)] = zero16
        return 0

    lax.fori_loop(0, DTOT // (8 * L), _zero, 0)

    slots = ((rv0, cv0, uv0, sg0, su0), (rv1, cv1, uv1, sg1, su1))

    def _start(c, si):
        rv, cv, _, sg, _ = slots[si]

        @pl.when(c < NCHUNK_A)
        def _():
            pltpu.async_copy(phi.at[ir_v.at[pl.ds(c * CHUNK_A, CHUNK_A)]], rv, sg)
            pltpu.async_copy(phi.at[ic_v.at[pl.ds(c * CHUNK_A, CHUNK_A)]], cv, sg)

    def _compute(c, si):
        rv, cv, uv, sg, su = slots[si]
        pltpu.make_async_copy(phi.at[pl.ds(0, CHUNK_A)], rv, sg).wait()
        pltpu.make_async_copy(phi.at[pl.ds(0, CHUNK_A)], cv, sg).wait()

        @pl.when(c >= 2)
        def _():
            pltpu.make_async_copy(uv, u_out.at[pl.ds(0, UCHUNK)], su).wait()

        def _group(g, _):
            eidx = g * L + lanes
            accs = []
            for k in range(N_COMS):
                a = zero16
                for d in range(k * 32, (k + 1) * 32):
                    dsplat = jnp.full((L,), d, jnp.int32)
                    a = a + (plsc.load_gather(rv, [eidx, dsplat])
                             * plsc.load_gather(cv, [eidx, dsplat]))
                accs.append(a)
            m = jnp.maximum(jnp.maximum(accs[0], accs[1]),
                            jnp.maximum(accs[2], accs[3]))
            ps = [jnp.exp(a - m) for a in accs]
            t = (ps[0] + ps[1]) + (ps[2] + ps[3])
            rvec = plsc.load_gather(ir_v, [c * CHUNK_A + eidx])
            for k in range(N_COMS):
                u = jnp.exp((ps[k] + ps[k]) / t)
                uv[pl.ds(k * CHUNK_A + g * L, L)] = u
                plsc.addupdate_scatter(acc, [rvec + (k * NBINS)], u)
            return 0

        lax.fori_loop(0, GROUPS_A, _group, 0)
        pltpu.async_copy(uv, u_out.at[pl.ds(ubase_w + c * UCHUNK, UCHUNK)], su)

    _start(0, 0)
    _start(1, 1)

    def _body(cc, _):
        c0 = cc * 2
        _compute(c0, 0)
        _start(c0 + 2, 0)
        _compute(c0 + 1, 1)
        _start(c0 + 3, 1)
        return 0

    lax.fori_loop(0, (NCHUNK_A - 1) // 2, _body, 0)
    _compute(NCHUNK_A - 1, 0)

    pltpu.make_async_copy(uv1, u_out.at[pl.ds(0, UCHUNK)], su1).wait()
    pltpu.make_async_copy(uv0, u_out.at[pl.ds(0, UCHUNK)], su0).wait()
    pltpu.sync_copy(acc, part_out.at[pl.ds(wid * DTOT, DTOT)])


@functools.partial(
    pl.kernel,
    out_type=jax.ShapeDtypeStruct((N_COMS * N_EDGES,), jnp.float32),
    mesh=_mesh,
    compiler_params=pltpu.CompilerParams(needs_layout_passes=False),
    scratch_types=(
        pltpu.VMEM((SLICE_B,), jnp.float32),
        pltpu.VMEM((SLICE_B,), jnp.float32),
        pltpu.VMEM((DTOT,), jnp.float32),
        pltpu.VMEM((CHUNK_B,), jnp.int32),
        pltpu.VMEM((SUB_B * UCHUNK,), jnp.float32),
        pltpu.VMEM((N_COMS * CHUNK_B,), jnp.float32),
        pltpu.VMEM_SHARED((DTOT,), jnp.float32),
        pltpu.SemaphoreType.DMA,
        pltpu.SemaphoreType.DMA,
    ),
)
def _phase_b(u_in, part, row, out, tmp_v, red_v, d_v, ri_v, u_v, o_v, shared,
             sem, so):
    sid = lax.axis_index("s")
    wid = _wid()
    base_w = wid * E_PER_W
    ubase_w = wid * (NCHUNK_A * UCHUNK)
    lanes = lax.iota(jnp.int32, L)
    zero16 = jnp.zeros((L,), jnp.float32)

    def _zero(j, _):
        for q in range(8):
            red_v[pl.ds((j * 8 + q) * L, L)] = zero16
        return 0

    lax.fori_loop(0, SLICE_B // (8 * L), _zero, 0)

    def _accum(p, _):
        pltpu.sync_copy(part.at[pl.ds(p * DTOT + sid * SLICE_B, SLICE_B)], tmp_v)

        def _add(j, _):
            for q in range(8):
                s = pl.ds((j * 8 + q) * L, L)
                red_v[s] = red_v[s] + tmp_v[s]
            return 0

        lax.fori_loop(0, SLICE_B // (8 * L), _add, 0)
        return 0

    lax.fori_loop(0, NW, _accum, 0)
    pltpu.sync_copy(red_v, shared.at[pl.ds(sid * SLICE_B, SLICE_B)])
    plsc.subcore_barrier()
    pltpu.sync_copy(shared, d_v)

    def _chunk(c, _):
        base = base_w + c * CHUNK_B
        pltpu.sync_copy(row.at[pl.ds(base, CHUNK_B)], ri_v)
        pltpu.sync_copy(
            u_in.at[pl.ds(ubase_w + c * (SUB_B * UCHUNK), SUB_B * UCHUNK)], u_v)

        @pl.when(c >= 1)
        def _():
            for k in range(N_COMS):
                pltpu.make_async_copy(
                    o_v.at[pl.ds(k * CHUNK_B, CHUNK_B)],
                    out.at[pl.ds(0, CHUNK_B)], so).wait()

        def _sub(l, _):
            for m in range(GROUPS_A):
                eidx = (l * GROUPS_A + m) * L + lanes
                rvec = plsc.load_gather(ri_v, [eidx])
                for k in range(N_COMS):
                    dk = plsc.load_gather(d_v, [rvec + (k * NBINS)])
                    u = u_v[pl.ds(l * UCHUNK + k * CHUNK_A + m * L, L)]
                    o_v[pl.ds(k * CHUNK_B + (l * GROUPS_A + m) * L, L)] = u / dk
            return 0

        lax.fori_loop(0, SUB_B, _sub, 0)
        for k in range(N_COMS):
            pltpu.async_copy(
                o_v.at[pl.ds(k * CHUNK_B, CHUNK_B)],
                out.at[pl.ds(k * N_EDGES + base, CHUNK_B)], so)
        return 0

    lax.fori_loop(0, NCHUNK_B, _chunk, 0)
    for k in range(N_COMS):
        pltpu.make_async_copy(
            o_v.at[pl.ds(k * CHUNK_B, CHUNK_B)],
            out.at[pl.ds(0, CHUNK_B)], so).wait()


def kernel(phi, edge_index):
    row = edge_index[0]
    col = edge_index[1]
    u, part = _phase_a(phi, row, col)
    return _phase_b(u, part, row).reshape(N_COMS, N_EDGES)
